# baseline (speedup 1.0000x reference)
import jax
import jax.numpy as jnp
from jax.experimental import pallas as pl
from jax.experimental.pallas import tpu as pltpu


_N_CHUNKS = 16
_N_SEED = 4


def _make_kernel(batch, seq_len, head_dim):
    ch = seq_len // _N_CHUNKS

    def _rot_dma_kernel(pos_ref, out_ref, vmem, rot_v, in_sem, out_sems):
        def start_out(i):
            for b in range(batch):
                pltpu.make_async_copy(
                    vmem.at[pl.ds(i * ch, ch)],
                    out_ref.at[b, pl.ds(i * ch, ch)],
                    out_sems.at[b],
                ).start()

        rot_cp = pltpu.make_async_copy(
            pos_ref.at[pl.ds(_N_SEED * ch, 1)], rot_v, in_sem
        )
        rot_cp.start()
        seed = pltpu.make_async_copy(
            pos_ref.at[pl.ds(0, _N_SEED * ch)],
            vmem.at[pl.ds(0, _N_SEED * ch)],
            in_sem,
        )
        seed.start()
        rot_cp.wait()
        seed.wait()
        for i in range(_N_SEED):
            start_out(i)

        col = jax.lax.broadcasted_iota(jnp.int32, (1, head_dim), 1)
        even = (col & 1) == 0
        t = rot_v[...]
        cosd = jnp.where(even, pltpu.roll(t, head_dim - 1, 1), t)
        ssind = jnp.where(even, t, -pltpu.roll(t, 1, 1))
        even_rows = jnp.broadcast_to(even, (ch, head_dim))

        for i in range(_N_SEED, _N_CHUNKS):
            prev = vmem[pl.ds((i - _N_SEED) * ch, ch), :]
            partner = jnp.where(
                even_rows,
                pltpu.roll(prev, head_dim - 1, 1),
                pltpu.roll(prev, 1, 1),
            )
            vmem[pl.ds(i * ch, ch), :] = prev * cosd + partner * ssind
            start_out(i)

        for i in range(_N_CHUNKS):
            for b in range(batch):
                pltpu.make_async_copy(
                    vmem.at[pl.ds(i * ch, ch)],
                    out_ref.at[b, pl.ds(i * ch, ch)],
                    out_sems.at[b],
                ).wait()

    return _rot_dma_kernel


def kernel(x, pos_embedding):
    batch, seq_len, head_dim = x.shape
    return pl.pallas_call(
        _make_kernel(batch, seq_len, head_dim),
        in_specs=[pl.BlockSpec(memory_space=pl.ANY)],
        out_specs=pl.BlockSpec(memory_space=pl.ANY),
        out_shape=jax.ShapeDtypeStruct(
            (batch, seq_len, head_dim), pos_embedding.dtype
        ),
        scratch_shapes=[
            pltpu.VMEM((seq_len, head_dim), pos_embedding.dtype),
            pltpu.VMEM((1, head_dim), pos_embedding.dtype),
            pltpu.SemaphoreType.DMA,
            pltpu.SemaphoreType.DMA((batch,)),
        ],
    )(pos_embedding)


# speedup vs baseline: 1.0314x; 1.0314x over previous
import jax
import jax.numpy as jnp
from jax.experimental import pallas as pl
from jax.experimental.pallas import tpu as pltpu


_N_CHUNKS = 16
_N_SEED = 2


def _make_kernel(batch, seq_len, head_dim):
    ch = seq_len // _N_CHUNKS

    def _rot_dma_kernel(pos_ref, out_ref, vmem, rot_v, in_sem, out_sems):
        def start_out(i):
            for b in range(batch):
                pltpu.make_async_copy(
                    vmem.at[pl.ds(i * ch, ch)],
                    out_ref.at[b, pl.ds(i * ch, ch)],
                    out_sems.at[b],
                ).start()

        rot_cp = pltpu.make_async_copy(
            pos_ref.at[pl.ds(_N_SEED * ch, 1)], rot_v, in_sem
        )
        rot_cp.start()
        seed = pltpu.make_async_copy(
            pos_ref.at[pl.ds(0, _N_SEED * ch)],
            vmem.at[pl.ds(0, _N_SEED * ch)],
            in_sem,
        )
        seed.start()
        rot_cp.wait()
        seed.wait()
        for i in range(_N_SEED):
            start_out(i)

        col = jax.lax.broadcasted_iota(jnp.int32, (1, head_dim), 1)
        even = (col & 1) == 0
        t = rot_v[...]
        cosd = jnp.where(even, pltpu.roll(t, head_dim - 1, 1), t)
        ssind = jnp.where(even, t, -pltpu.roll(t, 1, 1))
        even_rows = jnp.broadcast_to(even, (ch, head_dim))

        for i in range(_N_SEED, _N_CHUNKS):
            prev = vmem[pl.ds((i - _N_SEED) * ch, ch), :]
            partner = jnp.where(
                even_rows,
                pltpu.roll(prev, head_dim - 1, 1),
                pltpu.roll(prev, 1, 1),
            )
            vmem[pl.ds(i * ch, ch), :] = prev * cosd + partner * ssind
            start_out(i)

        for i in range(_N_CHUNKS):
            for b in range(batch):
                pltpu.make_async_copy(
                    vmem.at[pl.ds(i * ch, ch)],
                    out_ref.at[b, pl.ds(i * ch, ch)],
                    out_sems.at[b],
                ).wait()

    return _rot_dma_kernel


def kernel(x, pos_embedding):
    batch, seq_len, head_dim = x.shape
    return pl.pallas_call(
        _make_kernel(batch, seq_len, head_dim),
        in_specs=[pl.BlockSpec(memory_space=pl.ANY)],
        out_specs=pl.BlockSpec(memory_space=pl.ANY),
        out_shape=jax.ShapeDtypeStruct(
            (batch, seq_len, head_dim), pos_embedding.dtype
        ),
        scratch_shapes=[
            pltpu.VMEM((seq_len, head_dim), pos_embedding.dtype),
            pltpu.VMEM((1, head_dim), pos_embedding.dtype),
            pltpu.SemaphoreType.DMA,
            pltpu.SemaphoreType.DMA((batch,)),
        ],
    )(pos_embedding)
